# baseline (speedup 1.0000x reference)
import functools

import jax
import jax.numpy as jnp
from jax import lax
from jax.experimental import pallas as pl
from jax.experimental.pallas import tpu as pltpu
from jax.experimental.pallas import tpu_sc as plsc

_N_ATOMS = 100000
_NUM_WORKERS = 32
_PER_W = 3136
_LAST_W = _N_ATOMS - 31 * _PER_W
_N_SPECIES = 120
_LANES = 16


def _sc_body(x_hbm, z_hbm, tab_hbm, out_hbm, tab_v, x_v, z_v, o_v,
             sem_a, sem_b, sem_o):
    wid = lax.axis_index("s") * 2 + lax.axis_index("c")
    base = wid * _PER_W

    def gather_loop(lo, count, unroll):
        step = unroll * _LANES

        def body(i, carry):
            off = lo + i * step
            for j in range(unroll):
                o = off + j * _LANES
                z = z_v[pl.ds(o, _LANES)]
                s = plsc.load_gather(tab_v, [z])
                o_v[pl.ds(o, _LANES)] = x_v[pl.ds(o, _LANES)] - s
            return carry

        lax.fori_loop(0, count // step, body, 0)

    def work(size, unroll):
        half = size // 2
        cp_t = pltpu.make_async_copy(tab_hbm, tab_v, sem_a)
        cp_x0 = pltpu.make_async_copy(
            x_hbm.at[pl.ds(base, half)], x_v.at[pl.ds(0, half)], sem_a)
        cp_z0 = pltpu.make_async_copy(
            z_hbm.at[pl.ds(base, half)], z_v.at[pl.ds(0, half)], sem_a)
        cp_x1 = pltpu.make_async_copy(
            x_hbm.at[pl.ds(base + half, half)],
            x_v.at[pl.ds(half, half)], sem_b)
        cp_z1 = pltpu.make_async_copy(
            z_hbm.at[pl.ds(base + half, half)],
            z_v.at[pl.ds(half, half)], sem_b)
        cp_t.start()
        cp_x0.start()
        cp_z0.start()
        cp_x1.start()
        cp_z1.start()
        cp_t.wait()
        cp_x0.wait()
        cp_z0.wait()
        gather_loop(0, half, unroll)
        cp_o0 = pltpu.make_async_copy(
            o_v.at[pl.ds(0, half)], out_hbm.at[pl.ds(base, half)], sem_o)
        cp_o0.start()
        cp_x1.wait()
        cp_z1.wait()
        gather_loop(half, half, unroll)
        cp_o1 = pltpu.make_async_copy(
            o_v.at[pl.ds(half, half)],
            out_hbm.at[pl.ds(base + half, half)], sem_o)
        cp_o1.start()
        cp_o0.wait()
        cp_o1.wait()

    @pl.when(wid < _NUM_WORKERS - 1)
    def _():
        work(_PER_W, 7)

    @pl.when(wid == _NUM_WORKERS - 1)
    def _():
        work(_LAST_W, 3)


_sc_call = functools.partial(
    pl.kernel,
    out_type=jax.ShapeDtypeStruct((_N_ATOMS,), jnp.float32),
    mesh=plsc.VectorSubcoreMesh(core_axis_name="c", subcore_axis_name="s"),
    compiler_params=pltpu.CompilerParams(
        needs_layout_passes=False, skip_device_barrier=True),
    scratch_types=[
        pltpu.VMEM((_N_SPECIES,), jnp.float32),
        pltpu.VMEM((_PER_W,), jnp.float32),
        pltpu.VMEM((_PER_W,), jnp.int32),
        pltpu.VMEM((_PER_W,), jnp.float32),
        pltpu.SemaphoreType.DMA,
        pltpu.SemaphoreType.DMA,
        pltpu.SemaphoreType.DMA,
    ],
)(_sc_body)


@jax.jit
def kernel(x, Z, shift):
    return _sc_call(x, Z, shift.reshape(_N_SPECIES))


# speedup vs baseline: 1.0495x; 1.0495x over previous
import functools

import jax
import jax.numpy as jnp
from jax import lax
from jax.experimental import pallas as pl
from jax.experimental.pallas import tpu as pltpu
from jax.experimental.pallas import tpu_sc as plsc

_N_ATOMS = 100000
_NUM_WORKERS = 32
_PER_W = 3136
_LAST_W = _N_ATOMS - 31 * _PER_W
_N_SPECIES = 120
_LANES = 16


def _sc_body(x_hbm, z_hbm, tab_hbm, out_hbm, tab_v, x_v, z_v, o_v,
             sem_a, sem_b, sem_o):
    wid = lax.axis_index("s") * 2 + lax.axis_index("c")
    base = wid * _PER_W

    def work(size, unroll):
        cp_t = pltpu.make_async_copy(tab_hbm, tab_v, sem_a)
        cp_x = pltpu.make_async_copy(
            x_hbm.at[pl.ds(base, size)], x_v.at[pl.ds(0, size)], sem_a)
        cp_z = pltpu.make_async_copy(
            z_hbm.at[pl.ds(base, size)], z_v.at[pl.ds(0, size)], sem_a)
        cp_t.start()
        cp_x.start()
        cp_z.start()
        cp_t.wait()
        cp_x.wait()
        cp_z.wait()

        @plsc.parallel_loop(0, size, step=_LANES, unroll=unroll)
        def body(o):
            z = z_v[pl.ds(o, _LANES)]
            s = plsc.load_gather(tab_v, [z])
            o_v[pl.ds(o, _LANES)] = x_v[pl.ds(o, _LANES)] - s

        pltpu.sync_copy(o_v.at[pl.ds(0, size)], out_hbm.at[pl.ds(base, size)])

    @pl.when(wid < _NUM_WORKERS - 1)
    def _():
        work(_PER_W, 4)
    @pl.when(wid == _NUM_WORKERS - 1)
    def _():
        work(_LAST_W, 4)


_sc_call = functools.partial(
    pl.kernel,
    out_type=jax.ShapeDtypeStruct((_N_ATOMS,), jnp.float32),
    mesh=plsc.VectorSubcoreMesh(core_axis_name="c", subcore_axis_name="s"),
    compiler_params=pltpu.CompilerParams(
        needs_layout_passes=False, skip_device_barrier=True),
    scratch_types=[
        pltpu.VMEM((_N_SPECIES,), jnp.float32),
        pltpu.VMEM((_PER_W,), jnp.float32),
        pltpu.VMEM((_PER_W,), jnp.int32),
        pltpu.VMEM((_PER_W,), jnp.float32),
        pltpu.SemaphoreType.DMA,
        pltpu.SemaphoreType.DMA,
        pltpu.SemaphoreType.DMA,
    ],
)(_sc_body)


@jax.jit
def kernel(x, Z, shift):
    return _sc_call(x, Z, shift.reshape(_N_SPECIES))
